# baseline (speedup 1.0000x reference)
import functools

import jax
import jax.numpy as jnp
from jax.experimental import pallas as pl
from jax.experimental.pallas import tpu as pltpu


def _phase_a_kernel(x_ref, adj_ref, w1_ref, b1_ref, h_ref, adjb_ref, s_ref):
    t = pl.program_id(0)

    @pl.when(t == 0)
    def _():
        s_ref[...] = jnp.dot(
            x_ref[...].astype(jnp.bfloat16),
            w1_ref[...].astype(jnp.bfloat16),
            preferred_element_type=jnp.float32).astype(jnp.bfloat16)

    a16 = adj_ref[...].astype(jnp.bfloat16)
    adjb_ref[...] = a16
    t1 = jnp.dot(a16, s_ref[...], preferred_element_type=jnp.float32)
    h_ref[...] = jnp.maximum(t1 + b1_ref[...], 0.0).astype(jnp.bfloat16)


def _phase_b_kernel(adjb_ref, h_ref, w2_ref, b2_ref, o_ref):
    g = jnp.dot(adjb_ref[...], h_ref[...],
                preferred_element_type=jnp.float32)
    z = jnp.dot(g, w2_ref[...],
                preferred_element_type=jnp.float32) + b2_ref[...]
    o_ref[0, :, :] = jnp.sum(jnp.maximum(z, 0.0), axis=0, keepdims=True)


def kernel(x, adj, W1, b1, W2, b2):
    batch, n, nfeat = x.shape
    nhid = W1.shape[1]
    x2 = x.reshape(n, nfeat)
    adj2 = adj.reshape(n, n)

    bi = 400
    ni = n // bi

    h, adjb = pl.pallas_call(
        _phase_a_kernel,
        grid=(ni,),
        in_specs=[
            pl.BlockSpec((n, nfeat), lambda t: (0, 0)),
            pl.BlockSpec((bi, n), lambda t: (t, 0)),
            pl.BlockSpec((nfeat, nhid), lambda t: (0, 0)),
            pl.BlockSpec((1, nhid), lambda t: (0, 0)),
        ],
        out_specs=[
            pl.BlockSpec((bi, nhid), lambda t: (t, 0)),
            pl.BlockSpec((bi, n), lambda t: (t, 0)),
        ],
        out_shape=[
            jax.ShapeDtypeStruct((n, nhid), jnp.bfloat16),
            jax.ShapeDtypeStruct((n, n), jnp.bfloat16),
        ],
        scratch_shapes=[pltpu.VMEM((n, nhid), jnp.bfloat16)],
        compiler_params=pltpu.CompilerParams(
            dimension_semantics=("arbitrary",)),
    )(x2, adj2, W1, b1.reshape(1, nhid))

    partials = pl.pallas_call(
        _phase_b_kernel,
        grid=(ni,),
        in_specs=[
            pl.BlockSpec((bi, n), lambda t: (t, 0)),
            pl.BlockSpec((n, nhid), lambda t: (0, 0)),
            pl.BlockSpec((nhid, nfeat), lambda t: (0, 0)),
            pl.BlockSpec((1, nfeat), lambda t: (0, 0)),
        ],
        out_specs=pl.BlockSpec((1, 1, nfeat), lambda t: (t, 0, 0)),
        out_shape=jax.ShapeDtypeStruct((ni, 1, nfeat), jnp.float32),
        compiler_params=pltpu.CompilerParams(
            dimension_semantics=("arbitrary",)),
    )(adjb, h, W2, b2.reshape(1, nfeat))

    return (jnp.sum(partials) / (n * nfeat)).reshape(batch)


# speedup vs baseline: 1.1196x; 1.1196x over previous
import functools

import jax
import jax.numpy as jnp
from jax.experimental import pallas as pl
from jax.experimental.pallas import tpu as pltpu


def _fused_kernel(x_ref, adj_ref, w1_ref, b1_ref, w2_ref, b2_ref,
                  o_ref, s_ref, h_ref, acc_ref, *, ni, scale):
    t = pl.program_id(0)

    @pl.when(t == 0)
    def _():
        s_ref[...] = jnp.dot(
            x_ref[...].astype(jnp.bfloat16),
            w1_ref[...].astype(jnp.bfloat16),
            preferred_element_type=jnp.float32).astype(jnp.bfloat16)

    @pl.when(t < ni)
    def _():
        t1 = jnp.dot(adj_ref[...].astype(jnp.bfloat16), s_ref[...],
                     preferred_element_type=jnp.float32)
        bi = adj_ref.shape[0]
        h_ref[pl.ds(t * bi, bi), :] = jnp.maximum(
            t1 + b1_ref[...], 0.0).astype(jnp.bfloat16)

    @pl.when(t == ni)
    def _():
        acc_ref[...] = jnp.zeros_like(acc_ref)

    @pl.when(t >= ni)
    def _():
        g = jnp.dot(adj_ref[...].astype(jnp.bfloat16), h_ref[...],
                    preferred_element_type=jnp.float32)
        z = jnp.dot(g, w2_ref[...],
                    preferred_element_type=jnp.float32) + b2_ref[...]
        z = jnp.maximum(z, 0.0)
        acc_ref[...] += jnp.sum(z, axis=0, keepdims=True)

    @pl.when(t == 2 * ni - 1)
    def _():
        o_ref[0, :] = jnp.sum(acc_ref[...], axis=1) * scale


def kernel(x, adj, W1, b1, W2, b2):
    batch, n, nfeat = x.shape
    nhid = W1.shape[1]
    x2 = x.reshape(n, nfeat)
    adj2 = adj.reshape(n, n)

    bi = 400
    ni = n // bi

    def adj_idx(t):
        return (jnp.where(t < ni, t, 2 * ni - 1 - t), 0)

    out = pl.pallas_call(
        functools.partial(_fused_kernel, ni=ni, scale=1.0 / (n * nfeat)),
        grid=(2 * ni,),
        in_specs=[
            pl.BlockSpec((n, nfeat), lambda t: (0, 0)),
            pl.BlockSpec((bi, n), adj_idx),
            pl.BlockSpec((nfeat, nhid), lambda t: (0, 0)),
            pl.BlockSpec((1, nhid), lambda t: (0, 0)),
            pl.BlockSpec((nhid, nfeat), lambda t: (0, 0)),
            pl.BlockSpec((1, nfeat), lambda t: (0, 0)),
        ],
        out_specs=pl.BlockSpec((1, 1), lambda t: (0, 0)),
        out_shape=jax.ShapeDtypeStruct((1, 1), jnp.float32),
        scratch_shapes=[
            pltpu.VMEM((n, nhid), jnp.bfloat16),
            pltpu.VMEM((n, nhid), jnp.bfloat16),
            pltpu.VMEM((1, nfeat), jnp.float32),
        ],
        compiler_params=pltpu.CompilerParams(
            dimension_semantics=("arbitrary",)),
    )(x2, adj2, W1, b1.reshape(1, nhid), W2, b2.reshape(1, nfeat))

    return out.reshape(batch)
